# Initial kernel scaffold; baseline (speedup 1.0000x reference)
import jax
import jax.numpy as jnp
from jax.experimental import pallas as pl


def kernel(batch_data, item_embeddings):
    raise NotImplementedError("write your pallas kernel here")


# speedup vs baseline: 1.6841x; 1.6841x over previous
import functools

import jax
import jax.numpy as jnp
from jax import lax
from jax.experimental import pallas as pl
from jax.experimental.pallas import tpu as pltpu
from jax.experimental.pallas import tpu_sc as plsc

_IDX_W = 128


@functools.lru_cache(maxsize=None)
def _make_gather(n_rows, vocab, d):
    info = plsc.get_sparse_core_info()
    nw = info.num_cores * info.num_subcores
    assert n_rows % nw == 0
    rows_per_w = n_rows // nw

    mesh = plsc.VectorSubcoreMesh(core_axis_name="c", subcore_axis_name="s")

    @functools.partial(
        pl.kernel,
        mesh=mesh,
        out_type=jax.ShapeDtypeStruct((n_rows * _IDX_W, d), jnp.float32),
        scratch_types=[
            pltpu.VMEM((rows_per_w, _IDX_W), jnp.int32),
            pltpu.VMEM((_IDX_W, d), jnp.float32),
            pltpu.SemaphoreType.DMA,
        ],
        compiler_params=pltpu.CompilerParams(use_tc_tiling_on_sc=False),
    )
    def gather_kernel(idx_hbm, table_hbm, out_hbm, idx_v, rows_v, sem):
        wid = lax.axis_index("s") * info.num_cores + lax.axis_index("c")
        base_row = wid * rows_per_w
        pltpu.sync_copy(idx_hbm.at[pl.ds(base_row, rows_per_w)], idx_v)

        def body(g, carry):
            pltpu.async_copy(table_hbm.at[idx_v.at[g]], rows_v, sem).wait()
            pltpu.sync_copy(
                rows_v, out_hbm.at[pl.ds((base_row + g) * _IDX_W, _IDX_W)]
            )
            return carry

        lax.fori_loop(0, rows_per_w, body, 0)

    return gather_kernel


def kernel(batch_data, item_embeddings):
    batch, hist = batch_data.shape
    vocab, d = item_embeddings.shape
    flat = batch_data.reshape(-1).astype(jnp.int32)
    n_rows = flat.shape[0] // _IDX_W
    idx2d = flat.reshape(n_rows, _IDX_W)
    out = _make_gather(n_rows, vocab, d)(idx2d, item_embeddings)
    return out.reshape(batch, hist, d)


# speedup vs baseline: 1.8764x; 1.1142x over previous
import functools

import jax
import jax.numpy as jnp
from jax import lax
from jax.experimental import pallas as pl
from jax.experimental.pallas import tpu as pltpu
from jax.experimental.pallas import tpu_sc as plsc

_IDX_W = 128
_G = 2
_NBUF = 4


@functools.lru_cache(maxsize=None)
def _make_gather(n_rows, vocab, d):
    info = plsc.get_sparse_core_info()
    nw = info.num_cores * info.num_subcores
    assert n_rows % (nw * _G * _NBUF) == 0
    rows_per_w = n_rows // nw
    n_groups = rows_per_w // _G
    n_outer = n_groups // _NBUF
    grp_rows = _G * _IDX_W

    mesh = plsc.VectorSubcoreMesh(core_axis_name="c", subcore_axis_name="s")

    @functools.partial(
        pl.kernel,
        mesh=mesh,
        out_type=jax.ShapeDtypeStruct((n_rows * _IDX_W, d), jnp.float32),
        scratch_types=[
            pltpu.VMEM((rows_per_w, _IDX_W), jnp.int32),
            pltpu.VMEM((_NBUF, grp_rows, d), jnp.float32),
        ]
        + [pltpu.SemaphoreType.DMA] * (2 * _NBUF),
        compiler_params=pltpu.CompilerParams(use_tc_tiling_on_sc=False),
    )
    def gather_kernel(idx_hbm, table_hbm, out_hbm, idx_v, rows_v, *sems):
        gsem = sems[:_NBUF]
        osem = sems[_NBUF:]
        wid = lax.axis_index("s") * info.num_cores + lax.axis_index("c")
        base_row = wid * rows_per_w
        pltpu.sync_copy(idx_hbm.at[pl.ds(base_row, rows_per_w)], idx_v)

        def fire_gather(g, b):
            for j in range(_G):
                pltpu.async_copy(
                    table_hbm.at[idx_v.at[g * _G + j]],
                    rows_v.at[b, pl.ds(j * _IDX_W, _IDX_W)],
                    gsem[b],
                )

        def drain_gather(b):
            pltpu.make_async_copy(
                out_hbm.at[pl.ds(0, grp_rows)], rows_v.at[b], gsem[b]
            ).wait()

        def fire_out(g, b):
            pltpu.async_copy(
                rows_v.at[b],
                out_hbm.at[pl.ds((base_row + g * _G) * _IDX_W, grp_rows)],
                osem[b],
            )

        def drain_out(b):
            pltpu.make_async_copy(
                out_hbm.at[pl.ds(0, grp_rows)], rows_v.at[b], osem[b]
            ).wait()

        fire_gather(0, 0)

        def outer(t, carry):
            for b in range(_NBUF):
                g = t * _NBUF + b
                nb = (b + 1) % _NBUF

                @pl.when(g + 1 < n_groups)
                def _():
                    @pl.when(g >= _NBUF - 1)
                    def _():
                        drain_out(nb)

                    fire_gather(g + 1, nb)

                drain_gather(b)
                fire_out(g, b)
            return carry

        lax.fori_loop(0, n_outer, outer, 0)
        for b in range(_NBUF):
            drain_out(b)

    return gather_kernel


def kernel(batch_data, item_embeddings):
    batch, hist = batch_data.shape
    vocab, d = item_embeddings.shape
    flat = batch_data.reshape(-1).astype(jnp.int32)
    n_rows = flat.shape[0] // _IDX_W
    idx2d = flat.reshape(n_rows, _IDX_W)
    out = _make_gather(n_rows, vocab, d)(idx2d, item_embeddings)
    return out.reshape(batch, hist, d)
